# baseline (speedup 1.0000x reference)
import functools

import jax
import jax.numpy as jnp
from jax import lax
from jax.experimental import pallas as pl
from jax.experimental.pallas import tpu as pltpu
from jax.experimental.pallas import tpu_sc as plsc

NC, NS, L = 2, 16, 16
NW = NC * NS
N = 1_000_000
C = 8_000
NCHUNK = N // C
GRP = C // L
FULL = NCHUNK // NW
REM = NCHUNK % NW
MAXC = FULL + 1

B = 65_536
NB = -(-N // B)
TZ = 1_520
M_PAD = N + NW * TZ
assert M_PAD >= NB * B

_mesh = plsc.VectorSubcoreMesh(core_axis_name="c", subcore_axis_name="s")
_params = pltpu.CompilerParams(needs_layout_passes=False)


@functools.partial(
    pl.kernel,
    out_type=jax.ShapeDtypeStruct((M_PAD,), jnp.float32),
    mesh=_mesh,
    compiler_params=_params,
    scratch_types=[
        pltpu.VMEM((128,), jnp.float32),
        [pltpu.VMEM((C,), jnp.int32) for _ in range(MAXC)],
        [pltpu.VMEM((C,), jnp.float32) for _ in range(MAXC)],
        pltpu.VMEM((TZ,), jnp.float32),
        pltpu.SemaphoreType.DMA,
        pltpu.SemaphoreType.DMA,
    ],
)
def _sc_gather_masses(z_hbm, tab_hbm, m_hbm, tab_v, z_v, m_v, zt_v, zsem, osem):
    wid = lax.axis_index("s") * NC + lax.axis_index("c")
    pltpu.sync_copy(tab_hbm, tab_v)
    has_extra = wid < REM

    copies = []
    for k in range(MAXC):
        c = wid + k * NW
        cp = pltpu.make_async_copy(z_hbm.at[pl.ds(c * C, C)], z_v[k], zsem)
        if k < FULL:
            cp.start()
        else:
            @pl.when(has_extra)
            def _(cp=cp):
                cp.start()
        copies.append(cp)

    @plsc.parallel_loop(0, TZ // L)
    def _(j):
        zt_v[pl.ds(j * L, L)] = jnp.zeros((L,), jnp.float32)

    tail_cp = pltpu.make_async_copy(
        zt_v, m_hbm.at[pl.ds(N + wid * TZ, TZ)], osem
    )
    tail_cp.start()

    out_copies = []
    for k in range(MAXC):
        c = wid + k * NW
        ocp = pltpu.make_async_copy(m_v[k], m_hbm.at[pl.ds(c * C, C)], osem)

        def _do(k=k, cp=copies[k], ocp=ocp):
            cp.wait()

            @plsc.parallel_loop(0, GRP, unroll=8)
            def _(g):
                z = z_v[k][pl.ds(g * L, L)]
                m_v[k][pl.ds(g * L, L)] = plsc.load_gather(tab_v, [z])

            ocp.start()

        if k < FULL:
            _do()
        else:
            pl.when(has_extra)(_do)
        out_copies.append(ocp)

    tail_cp.wait()
    for k in range(MAXC):
        if k < FULL:
            out_copies[k].wait()
        else:
            @pl.when(has_extra)
            def _(ocp=out_copies[k]):
                ocp.wait()


def _tc_fused_body(
    x_ref, y_ref, z_ref, m_ref, pos_ref, out_ref, ax_ref, ay_ref, az_ref, am_ref, com_ref
):
    p = pl.program_id(0)
    i = pl.program_id(1)

    @pl.when(jnp.logical_and(p == 0, i == 0))
    def _():
        ax_ref[...] = jnp.zeros_like(ax_ref)
        ay_ref[...] = jnp.zeros_like(ay_ref)
        az_ref[...] = jnp.zeros_like(az_ref)
        am_ref[...] = jnp.zeros_like(am_ref)

    @pl.when(p == 0)
    def _():
        m = m_ref[...]
        ax_ref[...] += m * x_ref[...]
        ay_ref[...] += m * y_ref[...]
        az_ref[...] += m * z_ref[...]
        am_ref[...] += m

    @pl.when(jnp.logical_and(p == 1, i == 0))
    def _():
        sm = jnp.sum(am_ref[...])
        com_ref[0:1, :] = jnp.full((1, 128), jnp.sum(ax_ref[...]) / sm)
        com_ref[1:2, :] = jnp.full((1, 128), jnp.sum(ay_ref[...]) / sm)
        com_ref[2:3, :] = jnp.full((1, 128), jnp.sum(az_ref[...]) / sm)

    @pl.when(p == 1)
    def _():
        out_ref[...] = pos_ref[...] - com_ref[0:3, 0:1]


_tc_fused = pl.pallas_call(
    _tc_fused_body,
    grid=(2, NB),
    in_specs=[
        pl.BlockSpec((B,), lambda p, i: (i * (1 - p),)),
        pl.BlockSpec((B,), lambda p, i: (i * (1 - p),)),
        pl.BlockSpec((B,), lambda p, i: (i * (1 - p),)),
        pl.BlockSpec((B,), lambda p, i: (i * (1 - p),)),
        pl.BlockSpec((3, B), lambda p, i: (0, i * p)),
    ],
    out_specs=pl.BlockSpec((3, B), lambda p, i: (0, i * p)),
    out_shape=jax.ShapeDtypeStruct((3, N), jnp.float32),
    scratch_shapes=[
        pltpu.VMEM((B,), jnp.float32),
        pltpu.VMEM((B,), jnp.float32),
        pltpu.VMEM((B,), jnp.float32),
        pltpu.VMEM((B,), jnp.float32),
        pltpu.VMEM((4, 128), jnp.float32),
    ],
)


def kernel(Z, position, atomic_masses):
    post = position.T
    tab = jnp.zeros((128,), jnp.float32).at[: atomic_masses.shape[0]].set(atomic_masses)
    m = _sc_gather_masses(Z, tab)
    outt = _tc_fused(post[0], post[1], post[2], m, post)
    return outt.T


# speedup vs baseline: 1.6322x; 1.6322x over previous
import functools

import jax
import jax.numpy as jnp
from jax import lax
from jax.experimental import pallas as pl
from jax.experimental.pallas import tpu as pltpu
from jax.experimental.pallas import tpu_sc as plsc

NC, NS, L = 2, 16, 16
NW = NC * NS
N = 1_000_000
C = 8_000
NCHUNK = N // C
GRP = C // L
FULL = NCHUNK // NW
REM = NCHUNK % NW
MAXC = FULL + 1

B = 65_536
NB = -(-N // B)
TZ = 1_520
M_PAD = N + NW * TZ
assert M_PAD >= NB * B

_mesh = plsc.VectorSubcoreMesh(core_axis_name="c", subcore_axis_name="s")
_params = pltpu.CompilerParams(needs_layout_passes=False)


@functools.partial(
    pl.kernel,
    out_type=jax.ShapeDtypeStruct((M_PAD,), jnp.float32),
    mesh=_mesh,
    compiler_params=_params,
    scratch_types=[
        pltpu.VMEM((128,), jnp.float32),
        [pltpu.VMEM((C,), jnp.int32) for _ in range(MAXC)],
        [pltpu.VMEM((C,), jnp.float32) for _ in range(MAXC)],
        pltpu.VMEM((TZ,), jnp.float32),
        pltpu.SemaphoreType.DMA,
        pltpu.SemaphoreType.DMA,
    ],
)
def _sc_gather_masses(z_hbm, tab_hbm, m_hbm, tab_v, z_v, m_v, zt_v, zsem, osem):
    wid = lax.axis_index("s") * NC + lax.axis_index("c")
    pltpu.sync_copy(tab_hbm, tab_v)
    has_extra = wid < REM

    copies = []
    for k in range(MAXC):
        c = wid + k * NW
        cp = pltpu.make_async_copy(z_hbm.at[pl.ds(c * C, C)], z_v[k], zsem)
        if k < FULL:
            cp.start()
        else:
            @pl.when(has_extra)
            def _(cp=cp):
                cp.start()
        copies.append(cp)

    @plsc.parallel_loop(0, TZ // L)
    def _(j):
        zt_v[pl.ds(j * L, L)] = jnp.zeros((L,), jnp.float32)

    tail_cp = pltpu.make_async_copy(
        zt_v, m_hbm.at[pl.ds(N + wid * TZ, TZ)], osem
    )
    tail_cp.start()

    out_copies = []
    for k in range(MAXC):
        c = wid + k * NW
        ocp = pltpu.make_async_copy(m_v[k], m_hbm.at[pl.ds(c * C, C)], osem)

        def _do(k=k, cp=copies[k], ocp=ocp):
            cp.wait()

            @plsc.parallel_loop(0, GRP, unroll=8)
            def _(g):
                z = z_v[k][pl.ds(g * L, L)]
                m_v[k][pl.ds(g * L, L)] = plsc.load_gather(tab_v, [z])

            ocp.start()

        if k < FULL:
            _do()
        else:
            pl.when(has_extra)(_do)
        out_copies.append(ocp)

    tail_cp.wait()
    for k in range(MAXC):
        if k < FULL:
            out_copies[k].wait()
        else:
            @pl.when(has_extra)
            def _(ocp=out_copies[k]):
                ocp.wait()


def _tc_fused_body(m_ref, pos_ref, out_ref, acc_ref, am_ref, com_ref):
    p = pl.program_id(0)
    i = pl.program_id(1)

    @pl.when(jnp.logical_and(p == 0, i == 0))
    def _():
        acc_ref[...] = jnp.zeros_like(acc_ref)
        am_ref[...] = jnp.zeros_like(am_ref)

    @pl.when(p == 0)
    def _():
        m1 = m_ref[...]
        mp = m1.reshape(1, B) * pos_ref[...]
        acc_ref[0:3, 0:1] += jnp.sum(mp, axis=1, keepdims=True)
        am_ref[...] += m1

    @pl.when(jnp.logical_and(p == 1, i == 0))
    def _():
        sm = jnp.sum(am_ref[...])
        com_ref[0:3, :] = jnp.broadcast_to(acc_ref[0:3, 0:1] / sm, (3, 128))

    @pl.when(p == 1)
    def _():
        out_ref[...] = pos_ref[...] - com_ref[0:3, 0:1]


_tc_fused = pl.pallas_call(
    _tc_fused_body,
    grid=(2, NB),
    in_specs=[
        pl.BlockSpec((B,), lambda p, i: (i * (1 - p),)),
        pl.BlockSpec((3, B), lambda p, i: (0, i)),
    ],
    out_specs=pl.BlockSpec((3, B), lambda p, i: (0, i * p)),
    out_shape=jax.ShapeDtypeStruct((3, N), jnp.float32),
    scratch_shapes=[
        pltpu.VMEM((4, 128), jnp.float32),
        pltpu.VMEM((B,), jnp.float32),
        pltpu.VMEM((4, 128), jnp.float32),
    ],
)


def kernel(Z, position, atomic_masses):
    post = position.T
    tab = jnp.zeros((128,), jnp.float32).at[: atomic_masses.shape[0]].set(atomic_masses)
    m = _sc_gather_masses(Z, tab)
    outt = _tc_fused(m, post)
    return outt.T


# speedup vs baseline: 1.8309x; 1.1218x over previous
import functools

import jax
import jax.numpy as jnp
from jax import lax
from jax.experimental import pallas as pl
from jax.experimental.pallas import tpu as pltpu
from jax.experimental.pallas import tpu_sc as plsc

NC, NS, L = 2, 16, 16
NW = NC * NS
N = 1_000_000
C = 8_000
NCHUNK = N // C
GRP = C // L
FULL = NCHUNK // NW
REM = NCHUNK % NW
MAXC = FULL + 1

B = 65_536
NB = -(-N // B)
TZ = 1_520
M_PAD = N + NW * TZ
assert M_PAD >= NB * B

_mesh = plsc.VectorSubcoreMesh(core_axis_name="c", subcore_axis_name="s")
_params = pltpu.CompilerParams(needs_layout_passes=False)


@functools.partial(
    pl.kernel,
    out_type=jax.ShapeDtypeStruct((M_PAD,), jnp.float32),
    mesh=_mesh,
    compiler_params=_params,
    scratch_types=[
        pltpu.VMEM((128,), jnp.float32),
        [pltpu.VMEM((C,), jnp.int32) for _ in range(MAXC)],
        [pltpu.VMEM((C,), jnp.float32) for _ in range(MAXC)],
        pltpu.VMEM((TZ,), jnp.float32),
        pltpu.SemaphoreType.DMA,
        pltpu.SemaphoreType.DMA,
    ],
)
def _sc_gather_masses(z_hbm, tab_hbm, m_hbm, tab_v, z_v, m_v, zt_v, zsem, osem):
    wid = lax.axis_index("s") * NC + lax.axis_index("c")
    pltpu.sync_copy(tab_hbm, tab_v)
    has_extra = wid < REM

    copies = []
    for k in range(MAXC):
        c = wid + k * NW
        cp = pltpu.make_async_copy(z_hbm.at[pl.ds(c * C, C)], z_v[k], zsem)
        if k < FULL:
            cp.start()
        else:
            @pl.when(has_extra)
            def _(cp=cp):
                cp.start()
        copies.append(cp)

    @plsc.parallel_loop(0, TZ // L)
    def _(j):
        zt_v[pl.ds(j * L, L)] = jnp.zeros((L,), jnp.float32)

    tail_cp = pltpu.make_async_copy(
        zt_v, m_hbm.at[pl.ds(N + wid * TZ, TZ)], osem
    )
    tail_cp.start()

    out_copies = []
    for k in range(MAXC):
        c = wid + k * NW
        ocp = pltpu.make_async_copy(m_v[k], m_hbm.at[pl.ds(c * C, C)], osem)

        def _do(k=k, cp=copies[k], ocp=ocp):
            cp.wait()

            @plsc.parallel_loop(0, GRP, unroll=8)
            def _(g):
                z = z_v[k][pl.ds(g * L, L)]
                m_v[k][pl.ds(g * L, L)] = plsc.load_gather(tab_v, [z])

            ocp.start()

        if k < FULL:
            _do()
        else:
            pl.when(has_extra)(_do)
        out_copies.append(ocp)

    tail_cp.wait()
    for k in range(MAXC):
        if k < FULL:
            out_copies[k].wait()
        else:
            @pl.when(has_extra)
            def _(ocp=out_copies[k]):
                ocp.wait()


LASTV = N - (NB - 1) * B
LASTT = LASTV - (LASTV % 128)
TAILB = (N - 1) // 128


def _tc_fused_body(m_ref, pos_hbm, ptail_ref, out_ref, big_ref, acc_ref, am_ref, com_ref, sem):
    p = pl.program_id(0)
    i = pl.program_id(1)

    def _copy(k):
        w = B if k < NB - 1 else LASTT
        return pltpu.make_async_copy(
            pos_hbm.at[:, pl.ds(k * B, w)], big_ref.at[k, :, pl.ds(0, w)], sem
        )

    @pl.when(jnp.logical_and(p == 0, i == 0))
    def _():
        acc_ref[...] = jnp.zeros_like(acc_ref)
        am_ref[...] = jnp.zeros_like(am_ref)
        for k in range(NB):
            _copy(k).start()

    for k in range(NB):
        @pl.when(jnp.logical_and(p == 0, i == k))
        def _(k=k):
            _copy(k).wait()

    def _accumulate(masked):
        if masked:
            big_ref[NB - 1, :, pl.ds(LASTT, 128)] = ptail_ref[...]
        m1 = m_ref[...]
        mp = m1.reshape(1, B) * big_ref[i]
        if masked:
            lane = lax.broadcasted_iota(jnp.int32, (3, B), 1)
            mp = jnp.where(lane < LASTV, mp, 0.0)
        acc_ref[0:3, 0:1] += jnp.sum(mp, axis=1, keepdims=True)
        am_ref[...] += m1

    pl.when(jnp.logical_and(p == 0, i < NB - 1))(lambda: _accumulate(False))
    pl.when(jnp.logical_and(p == 0, i == NB - 1))(lambda: _accumulate(True))

    @pl.when(jnp.logical_and(p == 1, i == 0))
    def _():
        sm = jnp.sum(am_ref[...])
        com_ref[0:3, :] = jnp.broadcast_to(acc_ref[0:3, 0:1] / sm, (3, 128))

    @pl.when(p == 1)
    def _():
        out_ref[...] = big_ref[i] - com_ref[0:3, 0:1]


_tc_fused = pl.pallas_call(
    _tc_fused_body,
    grid=(2, NB),
    in_specs=[
        pl.BlockSpec((B,), lambda p, i: (i * (1 - p),)),
        pl.BlockSpec(memory_space=pl.ANY),
        pl.BlockSpec((3, 128), lambda p, i: (0, TAILB)),
    ],
    out_specs=pl.BlockSpec((3, B), lambda p, i: (0, i * p)),
    out_shape=jax.ShapeDtypeStruct((3, N), jnp.float32),
    scratch_shapes=[
        pltpu.VMEM((NB, 3, B), jnp.float32),
        pltpu.VMEM((4, 128), jnp.float32),
        pltpu.VMEM((B,), jnp.float32),
        pltpu.VMEM((4, 128), jnp.float32),
        pltpu.SemaphoreType.DMA,
    ],
)


def kernel(Z, position, atomic_masses):
    post = position.T
    tab = jnp.zeros((128,), jnp.float32).at[: atomic_masses.shape[0]].set(atomic_masses)
    m = _sc_gather_masses(Z, tab)
    outt = _tc_fused(m, post, post)
    return outt.T


# speedup vs baseline: 1.8378x; 1.0037x over previous
import functools

import jax
import jax.numpy as jnp
from jax import lax
from jax.experimental import pallas as pl
from jax.experimental.pallas import tpu as pltpu
from jax.experimental.pallas import tpu_sc as plsc

NC, NS, L = 2, 16, 16
NW = NC * NS
N = 1_000_000
C = 8_000
NCHUNK = N // C
GRP = C // L
FULL = NCHUNK // NW
REM = NCHUNK % NW
MAXC = FULL + 1

B = 65_536
NB = -(-N // B)
TZ = 1_520
M_PAD = N + NW * TZ
assert M_PAD >= NB * B

_mesh = plsc.VectorSubcoreMesh(core_axis_name="c", subcore_axis_name="s")
_params = pltpu.CompilerParams(needs_layout_passes=False)


@functools.partial(
    pl.kernel,
    out_type=jax.ShapeDtypeStruct((M_PAD,), jnp.float32),
    mesh=_mesh,
    compiler_params=_params,
    scratch_types=[
        pltpu.VMEM((128,), jnp.float32),
        [pltpu.VMEM((C,), jnp.int32) for _ in range(MAXC)],
        [pltpu.VMEM((C,), jnp.float32) for _ in range(MAXC)],
        pltpu.VMEM((TZ,), jnp.float32),
        pltpu.SemaphoreType.DMA,
        pltpu.SemaphoreType.DMA,
    ],
)
def _sc_gather_masses(z_hbm, tab_hbm, m_hbm, tab_v, z_v, m_v, zt_v, zsem, osem):
    wid = lax.axis_index("s") * NC + lax.axis_index("c")
    pltpu.sync_copy(tab_hbm, tab_v.at[pl.ds(0, 119)])
    has_extra = wid < REM

    copies = []
    for k in range(MAXC):
        c = wid + k * NW
        cp = pltpu.make_async_copy(z_hbm.at[pl.ds(c * C, C)], z_v[k], zsem)
        if k < FULL:
            cp.start()
        else:
            @pl.when(has_extra)
            def _(cp=cp):
                cp.start()
        copies.append(cp)

    @plsc.parallel_loop(0, TZ // L)
    def _(j):
        zt_v[pl.ds(j * L, L)] = jnp.zeros((L,), jnp.float32)

    tail_cp = pltpu.make_async_copy(
        zt_v, m_hbm.at[pl.ds(N + wid * TZ, TZ)], osem
    )
    tail_cp.start()

    out_copies = []
    for k in range(MAXC):
        c = wid + k * NW
        ocp = pltpu.make_async_copy(m_v[k], m_hbm.at[pl.ds(c * C, C)], osem)

        def _do(k=k, cp=copies[k], ocp=ocp):
            cp.wait()

            @plsc.parallel_loop(0, GRP, unroll=8)
            def _(g):
                z = z_v[k][pl.ds(g * L, L)]
                m_v[k][pl.ds(g * L, L)] = plsc.load_gather(tab_v, [z])

            ocp.start()

        if k < FULL:
            _do()
        else:
            pl.when(has_extra)(_do)
        out_copies.append(ocp)

    tail_cp.wait()
    for k in range(MAXC):
        if k < FULL:
            out_copies[k].wait()
        else:
            @pl.when(has_extra)
            def _(ocp=out_copies[k]):
                ocp.wait()


LASTV = N - (NB - 1) * B
LASTT = LASTV - (LASTV % 128)
TAILB = (N - 1) // 128


def _tc_fused_body(m_ref, pos_hbm, ptail_ref, out_ref, big_ref, acc_ref, am_ref, com_ref, sem):
    p = pl.program_id(0)
    i = pl.program_id(1)

    def _copy(k):
        w = B if k < NB - 1 else LASTT
        return pltpu.make_async_copy(
            pos_hbm.at[:, pl.ds(k * B, w)], big_ref.at[k, :, pl.ds(0, w)], sem
        )

    @pl.when(jnp.logical_and(p == 0, i == 0))
    def _():
        acc_ref[...] = jnp.zeros_like(acc_ref)
        am_ref[...] = jnp.zeros_like(am_ref)
        for k in range(NB):
            _copy(k).start()

    for k in range(NB):
        @pl.when(jnp.logical_and(p == 0, i == k))
        def _(k=k):
            _copy(k).wait()

    def _accumulate(masked):
        if masked:
            big_ref[NB - 1, :, pl.ds(LASTT, 128)] = ptail_ref[...]
        m1 = m_ref[...]
        mp = m1.reshape(1, B) * big_ref[i]
        if masked:
            lane = lax.broadcasted_iota(jnp.int32, (3, B), 1)
            mp = jnp.where(lane < LASTV, mp, 0.0)
        acc_ref[0:3, 0:1] += jnp.sum(mp, axis=1, keepdims=True)
        am_ref[...] += m1

    pl.when(jnp.logical_and(p == 0, i < NB - 1))(lambda: _accumulate(False))
    pl.when(jnp.logical_and(p == 0, i == NB - 1))(lambda: _accumulate(True))

    @pl.when(jnp.logical_and(p == 1, i == 0))
    def _():
        sm = jnp.sum(am_ref[...])
        com_ref[0:3, :] = jnp.broadcast_to(acc_ref[0:3, 0:1] / sm, (3, 128))

    @pl.when(p == 1)
    def _():
        out_ref[...] = big_ref[i] - com_ref[0:3, 0:1]


_tc_fused = pl.pallas_call(
    _tc_fused_body,
    grid=(2, NB),
    in_specs=[
        pl.BlockSpec((B,), lambda p, i: (i * (1 - p),)),
        pl.BlockSpec(memory_space=pl.ANY),
        pl.BlockSpec((3, 128), lambda p, i: (0, TAILB)),
    ],
    out_specs=pl.BlockSpec((3, B), lambda p, i: (0, i * p)),
    out_shape=jax.ShapeDtypeStruct((3, N), jnp.float32),
    scratch_shapes=[
        pltpu.VMEM((NB, 3, B), jnp.float32),
        pltpu.VMEM((4, 128), jnp.float32),
        pltpu.VMEM((B,), jnp.float32),
        pltpu.VMEM((4, 128), jnp.float32),
        pltpu.SemaphoreType.DMA,
    ],
)


def kernel(Z, position, atomic_masses):
    post = position.T
    m = _sc_gather_masses(Z, atomic_masses)
    outt = _tc_fused(m, post, post)
    return outt.T


# speedup vs baseline: 2.0686x; 1.1256x over previous
import functools

import jax
import jax.numpy as jnp
from jax import lax
from jax.experimental import pallas as pl
from jax.experimental.pallas import tpu as pltpu
from jax.experimental.pallas import tpu_sc as plsc

NC, NS, L = 2, 16, 16
NW = NC * NS
N = 1_000_000
C = 8_000
NCHUNK = N // C
GRP = C // L
FULL = NCHUNK // NW
REM = NCHUNK % NW
MAXC = FULL + 1

B = 131_072
NB = -(-N // B)
TZ = 1_520
M_PAD = N + NW * TZ
assert M_PAD >= NB * B

_mesh = plsc.VectorSubcoreMesh(core_axis_name="c", subcore_axis_name="s")
_params = pltpu.CompilerParams(needs_layout_passes=False)


@functools.partial(
    pl.kernel,
    out_type=jax.ShapeDtypeStruct((M_PAD,), jnp.float32),
    mesh=_mesh,
    compiler_params=_params,
    scratch_types=[
        pltpu.VMEM((128,), jnp.float32),
        [pltpu.VMEM((C,), jnp.int32) for _ in range(MAXC)],
        [pltpu.VMEM((C,), jnp.float32) for _ in range(MAXC)],
        pltpu.VMEM((TZ,), jnp.float32),
        pltpu.SemaphoreType.DMA,
        pltpu.SemaphoreType.DMA,
    ],
)
def _sc_gather_masses(z_hbm, tab_hbm, m_hbm, tab_v, z_v, m_v, zt_v, zsem, osem):
    wid = lax.axis_index("s") * NC + lax.axis_index("c")
    pltpu.sync_copy(tab_hbm, tab_v.at[pl.ds(0, 119)])
    has_extra = wid < REM

    copies = []
    for k in range(MAXC):
        c = wid + k * NW
        cp = pltpu.make_async_copy(z_hbm.at[pl.ds(c * C, C)], z_v[k], zsem)
        if k < FULL:
            cp.start()
        else:
            @pl.when(has_extra)
            def _(cp=cp):
                cp.start()
        copies.append(cp)

    @plsc.parallel_loop(0, TZ // L)
    def _(j):
        zt_v[pl.ds(j * L, L)] = jnp.zeros((L,), jnp.float32)

    tail_cp = pltpu.make_async_copy(
        zt_v, m_hbm.at[pl.ds(N + wid * TZ, TZ)], osem
    )
    tail_cp.start()

    out_copies = []
    for k in range(MAXC):
        c = wid + k * NW
        ocp = pltpu.make_async_copy(m_v[k], m_hbm.at[pl.ds(c * C, C)], osem)

        def _do(k=k, cp=copies[k], ocp=ocp):
            cp.wait()

            @plsc.parallel_loop(0, GRP, unroll=8)
            def _(g):
                z = z_v[k][pl.ds(g * L, L)]
                m_v[k][pl.ds(g * L, L)] = plsc.load_gather(tab_v, [z])

            ocp.start()

        if k < FULL:
            _do()
        else:
            pl.when(has_extra)(_do)
        out_copies.append(ocp)

    tail_cp.wait()
    for k in range(MAXC):
        if k < FULL:
            out_copies[k].wait()
        else:
            @pl.when(has_extra)
            def _(ocp=out_copies[k]):
                ocp.wait()


LASTV = N - (NB - 1) * B
LASTT = LASTV - (LASTV % 128)
TAILB = (N - 1) // 128


def _tc_fused_body(m_ref, pos_hbm, ptail_ref, out_ref, big_ref, acc_ref, am_ref, com_ref, sem):
    p = pl.program_id(0)
    i = pl.program_id(1)

    def _copy(k):
        w = B if k < NB - 1 else LASTT
        return pltpu.make_async_copy(
            pos_hbm.at[:, pl.ds(k * B, w)], big_ref.at[k, :, pl.ds(0, w)], sem
        )

    @pl.when(jnp.logical_and(p == 0, i == 0))
    def _():
        acc_ref[...] = jnp.zeros_like(acc_ref)
        am_ref[...] = jnp.zeros_like(am_ref)
        for k in range(NB):
            _copy(k).start()

    for k in range(NB):
        @pl.when(jnp.logical_and(p == 0, i == k))
        def _(k=k):
            _copy(k).wait()

    def _accumulate(masked):
        if masked:
            big_ref[NB - 1, :, pl.ds(LASTT, 128)] = ptail_ref[...]
        m1 = m_ref[...]
        mp = m1.reshape(1, B) * big_ref[i]
        if masked:
            lane = lax.broadcasted_iota(jnp.int32, (3, B), 1)
            mp = jnp.where(lane < LASTV, mp, 0.0)
        acc_ref[0:3, 0:1] += jnp.sum(mp, axis=1, keepdims=True)
        am_ref[...] += m1

    pl.when(jnp.logical_and(p == 0, i < NB - 1))(lambda: _accumulate(False))
    pl.when(jnp.logical_and(p == 0, i == NB - 1))(lambda: _accumulate(True))

    @pl.when(jnp.logical_and(p == 1, i == 0))
    def _():
        sm = jnp.sum(am_ref[...])
        com_ref[0:3, :] = jnp.broadcast_to(acc_ref[0:3, 0:1] / sm, (3, 128))

    @pl.when(p == 1)
    def _():
        out_ref[...] = big_ref[i] - com_ref[0:3, 0:1]


_tc_fused = pl.pallas_call(
    _tc_fused_body,
    grid=(2, NB),
    in_specs=[
        pl.BlockSpec((B,), lambda p, i: (i * (1 - p),)),
        pl.BlockSpec(memory_space=pl.ANY),
        pl.BlockSpec((3, 128), lambda p, i: (0, TAILB)),
    ],
    out_specs=pl.BlockSpec((3, B), lambda p, i: (0, i * p)),
    out_shape=jax.ShapeDtypeStruct((3, N), jnp.float32),
    scratch_shapes=[
        pltpu.VMEM((NB, 3, B), jnp.float32),
        pltpu.VMEM((4, 128), jnp.float32),
        pltpu.VMEM((B,), jnp.float32),
        pltpu.VMEM((4, 128), jnp.float32),
        pltpu.SemaphoreType.DMA,
    ],
)


def kernel(Z, position, atomic_masses):
    post = position.T
    m = _sc_gather_masses(Z, atomic_masses)
    outt = _tc_fused(m, post, post)
    return outt.T


# speedup vs baseline: 2.1195x; 1.0246x over previous
import functools

import jax
import jax.numpy as jnp
from jax import lax
from jax.experimental import pallas as pl
from jax.experimental.pallas import tpu as pltpu
from jax.experimental.pallas import tpu_sc as plsc

NC, NS, L = 2, 16, 16
NW = NC * NS
N = 1_000_000
C = 8_000
NCHUNK = N // C
GRP = C // L
FULL = NCHUNK // NW
REM = NCHUNK % NW
MAXC = FULL + 1

B = 262_144
NB = -(-N // B)
TZ = 1_520
M_PAD = N + NW * TZ
assert M_PAD >= NB * B

_mesh = plsc.VectorSubcoreMesh(core_axis_name="c", subcore_axis_name="s")
_params = pltpu.CompilerParams(needs_layout_passes=False)


@functools.partial(
    pl.kernel,
    out_type=jax.ShapeDtypeStruct((M_PAD,), jnp.float32),
    mesh=_mesh,
    compiler_params=_params,
    scratch_types=[
        pltpu.VMEM((128,), jnp.float32),
        [pltpu.VMEM((C,), jnp.int32) for _ in range(MAXC)],
        [pltpu.VMEM((C,), jnp.float32) for _ in range(MAXC)],
        pltpu.VMEM((TZ,), jnp.float32),
        pltpu.SemaphoreType.DMA,
        pltpu.SemaphoreType.DMA,
    ],
)
def _sc_gather_masses(z_hbm, tab_hbm, m_hbm, tab_v, z_v, m_v, zt_v, zsem, osem):
    wid = lax.axis_index("s") * NC + lax.axis_index("c")
    pltpu.sync_copy(tab_hbm, tab_v.at[pl.ds(0, 119)])
    has_extra = wid < REM

    copies = []
    for k in range(MAXC):
        c = wid + k * NW
        cp = pltpu.make_async_copy(z_hbm.at[pl.ds(c * C, C)], z_v[k], zsem)
        if k < FULL:
            cp.start()
        else:
            @pl.when(has_extra)
            def _(cp=cp):
                cp.start()
        copies.append(cp)

    @plsc.parallel_loop(0, TZ // L)
    def _(j):
        zt_v[pl.ds(j * L, L)] = jnp.zeros((L,), jnp.float32)

    tail_cp = pltpu.make_async_copy(
        zt_v, m_hbm.at[pl.ds(N + wid * TZ, TZ)], osem
    )
    tail_cp.start()

    out_copies = []
    for k in range(MAXC):
        c = wid + k * NW
        ocp = pltpu.make_async_copy(m_v[k], m_hbm.at[pl.ds(c * C, C)], osem)

        def _do(k=k, cp=copies[k], ocp=ocp):
            cp.wait()

            @plsc.parallel_loop(0, GRP, unroll=8)
            def _(g):
                z = z_v[k][pl.ds(g * L, L)]
                m_v[k][pl.ds(g * L, L)] = plsc.load_gather(tab_v, [z])

            ocp.start()

        if k < FULL:
            _do()
        else:
            pl.when(has_extra)(_do)
        out_copies.append(ocp)

    tail_cp.wait()
    for k in range(MAXC):
        if k < FULL:
            out_copies[k].wait()
        else:
            @pl.when(has_extra)
            def _(ocp=out_copies[k]):
                ocp.wait()


LASTV = N - (NB - 1) * B
LASTT = LASTV - (LASTV % 128)
TAILB = (N - 1) // 128


def _tc_fused_body(m_ref, pos_hbm, ptail_ref, out_ref, big_ref, acc_ref, am_ref, com_ref, sem):
    p = pl.program_id(0)
    i = pl.program_id(1)

    def _copy(k):
        w = B if k < NB - 1 else LASTT
        return pltpu.make_async_copy(
            pos_hbm.at[:, pl.ds(k * B, w)], big_ref.at[k, :, pl.ds(0, w)], sem
        )

    @pl.when(jnp.logical_and(p == 0, i == 0))
    def _():
        acc_ref[...] = jnp.zeros_like(acc_ref)
        am_ref[...] = jnp.zeros_like(am_ref)
        for k in range(NB):
            _copy(k).start()

    for k in range(NB):
        @pl.when(jnp.logical_and(p == 0, i == k))
        def _(k=k):
            _copy(k).wait()

    def _accumulate(masked):
        if masked:
            big_ref[NB - 1, :, pl.ds(LASTT, 128)] = ptail_ref[...]
        m1 = m_ref[...]
        mp = m1.reshape(1, B) * big_ref[i]
        if masked:
            lane = lax.broadcasted_iota(jnp.int32, (3, B), 1)
            mp = jnp.where(lane < LASTV, mp, 0.0)
        acc_ref[0:3, 0:1] += jnp.sum(mp, axis=1, keepdims=True)
        am_ref[...] += m1

    pl.when(jnp.logical_and(p == 0, i < NB - 1))(lambda: _accumulate(False))
    pl.when(jnp.logical_and(p == 0, i == NB - 1))(lambda: _accumulate(True))

    @pl.when(jnp.logical_and(p == 1, i == 0))
    def _():
        sm = jnp.sum(am_ref[...])
        com_ref[0:3, :] = jnp.broadcast_to(acc_ref[0:3, 0:1] / sm, (3, 128))

    @pl.when(p == 1)
    def _():
        out_ref[...] = big_ref[i] - com_ref[0:3, 0:1]


_tc_fused = pl.pallas_call(
    _tc_fused_body,
    grid=(2, NB),
    in_specs=[
        pl.BlockSpec((B,), lambda p, i: (i * (1 - p),)),
        pl.BlockSpec(memory_space=pl.ANY),
        pl.BlockSpec((3, 128), lambda p, i: (0, TAILB)),
    ],
    out_specs=pl.BlockSpec((3, B), lambda p, i: (0, i * p)),
    out_shape=jax.ShapeDtypeStruct((3, N), jnp.float32),
    scratch_shapes=[
        pltpu.VMEM((NB, 3, B), jnp.float32),
        pltpu.VMEM((4, 128), jnp.float32),
        pltpu.VMEM((B,), jnp.float32),
        pltpu.VMEM((4, 128), jnp.float32),
        pltpu.SemaphoreType.DMA,
    ],
)


def kernel(Z, position, atomic_masses):
    post = position.T
    m = _sc_gather_masses(Z, atomic_masses)
    outt = _tc_fused(m, post, post)
    return outt.T
